# baseline (speedup 1.0000x reference)
import functools

import jax
import jax.numpy as jnp
from jax import lax
from jax.experimental import pallas as pl
from jax.experimental.pallas import tpu as pltpu
from jax.experimental.pallas import tpu_sc as plsc

_D0, _N, _D2, _C = 2, 1024, 16, 1024
_R = 128
_NC, _NS = 2, 16
_NW = _NC * _NS
_KPW = _N // _NW


def _copy_kernel(t_ref, o_ref):
    o_ref[...] = t_ref[...]


def _tc_copy(t):
    grid = (_D0, _N // _R)
    return pl.pallas_call(
        _copy_kernel,
        grid=grid,
        in_specs=[pl.BlockSpec((1, _R, _D2, _C), lambda j, i: (j, i, 0, 0))],
        out_specs=pl.BlockSpec((1, _R, _D2, _C), lambda j, i: (j, i, 0, 0)),
        out_shape=jax.ShapeDtypeStruct(t.shape, t.dtype),
    )(t)


@functools.partial(
    pl.kernel,
    out_type=(),
    mesh=plsc.VectorSubcoreMesh(core_axis_name="c", subcore_axis_name="s"),
    scratch_types=[
        pltpu.VMEM((_N,), jnp.int32),
        pltpu.VMEM((_D0 * _D2, _N), jnp.float32),
        pltpu.SemaphoreType.DMA,
    ],
)
def _sc_diag_scatter(out_ref, idx_hbm, vt_hbm, idx_v, v_v, sem):
    wid = lax.axis_index("s") * _NC + lax.axis_index("c")
    base_k = wid * _KPW
    pltpu.sync_copy(idx_hbm, idx_v)
    pltpu.sync_copy(vt_hbm, v_v)
    copies = []
    for g in range(_KPW // 16):
        idx16 = idx_v[pl.ds(base_k + g * 16, 16)]
        base16 = idx16 * (_D2 * _C + 1)
        for d0 in range(_D0):
            for d2 in range(_D2):
                off16 = base16 + (d0 * _N * _D2 * _C + d2 * _C)
                copies.append(pltpu.async_copy(
                    v_v.at[d0 * _D2 + d2, pl.ds(base_k + g * 16, 16)],
                    out_ref.at[off16], sem))
    for c in copies:
        c.wait()


@functools.partial(jax.jit, static_argnames=())
def kernel(t, idx, v):
    out = _tc_copy(t)
    out_ref = jax.new_ref(out.reshape(_D0 * _N * _D2 * _C))
    idx32 = idx.astype(jnp.int32)
    vt = v.transpose(1, 2, 0).reshape(_D0 * _D2, _N)
    _sc_diag_scatter(out_ref, idx32, vt)
    return out_ref[...].reshape(_D0, _N, _D2, _C)
